# baseline (speedup 1.0000x reference)
import functools

import jax
import jax.numpy as jnp
from jax import lax
from jax.experimental import pallas as pl
from jax.experimental.pallas import tpu as pltpu
from jax.experimental.pallas import tpu_sc as plsc

B = 128
N = 8192
M = 8192
L = 16
NSUB = 16
ROWS_PER_SUB = B // NSUB
CHUNKS = N // L
UNROLL = 8

_mesh = plsc.VectorSubcoreMesh(core_axis_name="c", subcore_axis_name="s",
                               num_cores=2, num_subcores=NSUB)
_params = pltpu.CompilerParams(needs_layout_passes=False)


def _row_argmax(row_ref):
    iota = lax.iota(jnp.int32, L)
    vmax0 = jnp.full((L,), -jnp.inf, jnp.float32)
    vchunk0 = jnp.zeros((L,), jnp.int32)

    def body(u, carry):
        vmax, vchunk = carry
        for k in range(UNROLL):
            c = u * UNROLL + k
            v = row_ref[pl.ds(c * L, L)]
            gt = v > vmax
            vmax = jnp.where(gt, v, vmax)
            vchunk = jnp.where(gt, jnp.full((L,), c, jnp.int32), vchunk)
        return vmax, vchunk

    vmax, vchunk = lax.fori_loop(0, CHUNKS // UNROLL, body, (vmax0, vchunk0))
    m = jnp.max(vmax)
    idx = vchunk * L + iota
    cand = jnp.where(vmax == m, idx, jnp.int32(2**30))
    return jnp.min(cand)


def _argmax_body(rec_hbm, sub_hbm, ri_hbm, si_hbm, buf0, buf1, res_buf,
                 sem0, sem1):
    c = lax.axis_index("c")
    s = lax.axis_index("s")
    bufs = [buf0, buf1]
    sems = [sem0, sem1]

    def run(mat_hbm, out_hbm):
        base = s * ROWS_PER_SUB
        lane = lax.iota(jnp.int32, L)
        res_vec = jnp.zeros((L,), jnp.int32)
        pltpu.async_copy(mat_hbm.at[base], bufs[0], sems[0])
        for j in range(ROWS_PER_SUB):
            if j + 1 < ROWS_PER_SUB:
                pltpu.async_copy(mat_hbm.at[base + j + 1], bufs[(j + 1) % 2],
                                 sems[(j + 1) % 2])
            pltpu.make_async_copy(mat_hbm.at[base + j], bufs[j % 2],
                                  sems[j % 2]).wait()
            am = _row_argmax(bufs[j % 2])
            res_vec = jnp.where(lane == j, jnp.full((L,), am, jnp.int32),
                                res_vec)
        res_buf[...] = res_vec
        pltpu.sync_copy(res_buf.at[pl.ds(0, ROWS_PER_SUB)],
                        out_hbm.at[pl.ds(base, ROWS_PER_SUB)])

    @pl.when(c == 0)
    def _():
        run(rec_hbm, ri_hbm)

    @pl.when(c == 1)
    def _():
        run(sub_hbm, si_hbm)


_argmax_call = functools.partial(
    pl.kernel,
    out_type=(
        jax.ShapeDtypeStruct((B,), jnp.int32),
        jax.ShapeDtypeStruct((B,), jnp.int32),
    ),
    mesh=_mesh,
    scratch_types=[
        pltpu.VMEM((N,), jnp.float32),
        pltpu.VMEM((N,), jnp.float32),
        pltpu.VMEM((L,), jnp.int32),
        pltpu.SemaphoreType.DMA,
        pltpu.SemaphoreType.DMA,
    ],
    compiler_params=_params,
)(_argmax_body)


def _gather_social_body(ri_hbm, si_hbm, vid_hbm, pref_hbm, struct_hbm,
                        tp_hbm, ts_hbm, th_hbm,
                        ri_v, si_v, vid_v, pidx, pvidx, sidx,
                        pi_v, pv_v, th_v, tp_v, ts_buf,
                        si_sh, pi_sh, sem0, sem1, sem2):
    c = lax.axis_index("c")
    s = lax.axis_index("s")
    iota = lax.iota(jnp.int32, L)

    @pl.when(jnp.logical_and(c == 0, s == 0))
    def _():
        pltpu.sync_copy(ri_hbm, ri_v)
        pltpu.sync_copy(si_hbm, si_v)
        pltpu.async_copy(vid_hbm.at[ri_v], vid_v, sem0).wait()
        for cc in range(B // L):
            sl = pl.ds(cc * L, L)
            riv = ri_v[sl]
            siv = si_v[sl]
            vv = vid_v[sl]
            rows = iota + cc * L
            pidx[sl] = siv * B + rows
            pvidx[sl] = vv * B + rows
            sidx[sl] = siv * N + riv
        cp0 = pltpu.async_copy(pref_hbm.at[pidx], pi_v, sem0)
        cp1 = pltpu.async_copy(pref_hbm.at[pvidx], pv_v, sem1)
        cp2 = pltpu.async_copy(struct_hbm.at[sidx], th_v, sem2)
        cp0.wait()
        cp1.wait()
        cp2.wait()
        for cc in range(B // L):
            sl = pl.ds(cc * L, L)
            tp_v[sl] = pi_v[sl] - pv_v[sl]
        pltpu.sync_copy(tp_v, tp_hbm)
        pltpu.sync_copy(th_v, th_hbm)
        pltpu.sync_copy(si_v, si_sh)
        pltpu.sync_copy(pi_v, pi_sh)

    plsc.subcore_barrier()

    @pl.when(c == 0)
    def _():
        pltpu.sync_copy(si_sh, si_v)
        pltpu.sync_copy(pi_sh, pi_v)
        base = s * ROWS_PER_SUB
        ts_vec = jnp.zeros((L,), jnp.float32)
        for j in range(ROWS_PER_SUB):
            i = base + j
            ivec = jnp.full((L,), i, jnp.int32)
            s_ib = plsc.load_gather(si_v, [ivec])
            p_ib = plsc.load_gather(pi_v, [ivec])
            acc = jnp.zeros((L,), jnp.float32)
            for cc in range(B // L):
                sl = pl.ds(cc * L, L)
                sik = si_v[sl]
                pik = pi_v[sl]
                idxv = iota + cc * L
                msk = jnp.logical_and(sik == s_ib, idxv != i)
                acc = acc + jnp.where(msk, jnp.minimum(pik, p_ib), 0.0)
            ts_j = jnp.sum(acc)
            ts_vec = jnp.where(iota == j, jnp.full((L,), ts_j, jnp.float32),
                               ts_vec)
        ts_buf[...] = ts_vec
        pltpu.sync_copy(ts_buf.at[pl.ds(0, ROWS_PER_SUB)],
                        ts_hbm.at[pl.ds(base, ROWS_PER_SUB)])


_gather_social_call = functools.partial(
    pl.kernel,
    out_type=(
        jax.ShapeDtypeStruct((B,), jnp.float32),
        jax.ShapeDtypeStruct((B,), jnp.float32),
        jax.ShapeDtypeStruct((B,), jnp.float32),
    ),
    mesh=_mesh,
    scratch_types=[
        pltpu.VMEM((B,), jnp.int32),
        pltpu.VMEM((B,), jnp.int32),
        pltpu.VMEM((B,), jnp.int32),
        pltpu.VMEM((B,), jnp.int32),
        pltpu.VMEM((B,), jnp.int32),
        pltpu.VMEM((B,), jnp.int32),
        pltpu.VMEM((B,), jnp.float32),
        pltpu.VMEM((B,), jnp.float32),
        pltpu.VMEM((B,), jnp.float32),
        pltpu.VMEM((B,), jnp.float32),
        pltpu.VMEM((L,), jnp.float32),
        pltpu.VMEM_SHARED((B,), jnp.int32),
        pltpu.VMEM_SHARED((B,), jnp.float32),
        pltpu.SemaphoreType.DMA,
        pltpu.SemaphoreType.DMA,
        pltpu.SemaphoreType.DMA,
    ],
    compiler_params=_params,
)(_gather_social_body)


def kernel(Recommended_m, Substitute_m, ItemGroups_m, Vid, VUU, KUU, Vscore,
           Kscore, preference, structure):
    del ItemGroups_m, VUU, KUU, Vscore, Kscore
    ri, si = _argmax_call(Recommended_m, Substitute_m)
    vid_s = Vid[100:]
    tp, ts, th = _gather_social_call(
        ri, si, vid_s, preference.reshape(-1), structure.reshape(-1))
    return tp[:, None], ts[:, None], th[:, None]


# speedup vs baseline: 6.0059x; 6.0059x over previous
import functools

import jax
import jax.numpy as jnp
from jax import lax
from jax.experimental import pallas as pl
from jax.experimental.pallas import tpu as pltpu
from jax.experimental.pallas import tpu_sc as plsc

B = 128
N = 8192
M = 8192
L = 16
NSUB = 16
ROWS_PER_SUB = B // NSUB
CHUNKS = N // L
UNROLL = 8

_mesh = plsc.VectorSubcoreMesh(core_axis_name="c", subcore_axis_name="s",
                               num_cores=2, num_subcores=NSUB)
_params = pltpu.CompilerParams(needs_layout_passes=False)


def _row_argmax(row_ref):
    iota = lax.iota(jnp.int32, L)
    vmax0 = jnp.full((L,), -jnp.inf, jnp.float32)
    vchunk0 = jnp.zeros((L,), jnp.int32)

    def body(u, carry):
        vmax, vchunk = carry
        for k in range(UNROLL):
            c = u * UNROLL + k
            v = row_ref[pl.ds(c * L, L)]
            gt = v > vmax
            vmax = jnp.where(gt, v, vmax)
            vchunk = jnp.where(gt, jnp.full((L,), c, jnp.int32), vchunk)
        return vmax, vchunk

    vmax, vchunk = lax.fori_loop(0, CHUNKS // UNROLL, body, (vmax0, vchunk0))
    m = jnp.max(vmax)
    idx = vchunk * L + iota
    cand = jnp.where(vmax == m, idx, jnp.int32(2**30))
    return jnp.min(cand)


def _argmax_body(rec_hbm, sub_hbm, ri_hbm, si_hbm, buf0, buf1, res_buf,
                 sem0, sem1):
    c = lax.axis_index("c")
    s = lax.axis_index("s")
    bufs = [buf0, buf1]
    sems = [sem0, sem1]

    def run(mat_hbm, out_hbm):
        base = s * ROWS_PER_SUB
        lane = lax.iota(jnp.int32, L)
        res_vec = jnp.zeros((L,), jnp.int32)
        pltpu.async_copy(mat_hbm.at[base], bufs[0], sems[0])
        for j in range(ROWS_PER_SUB):
            if j + 1 < ROWS_PER_SUB:
                pltpu.async_copy(mat_hbm.at[base + j + 1], bufs[(j + 1) % 2],
                                 sems[(j + 1) % 2])
            pltpu.make_async_copy(mat_hbm.at[base + j], bufs[j % 2],
                                  sems[j % 2]).wait()
            am = _row_argmax(bufs[j % 2])
            res_vec = jnp.where(lane == j, jnp.full((L,), am, jnp.int32),
                                res_vec)
        res_buf[...] = res_vec
        pltpu.sync_copy(res_buf.at[pl.ds(0, ROWS_PER_SUB)],
                        out_hbm.at[pl.ds(base, ROWS_PER_SUB)])

    @pl.when(c == 0)
    def _():
        run(rec_hbm, ri_hbm)

    @pl.when(c == 1)
    def _():
        run(sub_hbm, si_hbm)


_argmax_call = functools.partial(
    pl.kernel,
    out_type=(
        jax.ShapeDtypeStruct((B,), jnp.int32),
        jax.ShapeDtypeStruct((B,), jnp.int32),
    ),
    mesh=_mesh,
    scratch_types=[
        pltpu.VMEM((N,), jnp.float32),
        pltpu.VMEM((N,), jnp.float32),
        pltpu.VMEM((L,), jnp.int32),
        pltpu.SemaphoreType.DMA,
        pltpu.SemaphoreType.DMA,
    ],
    compiler_params=_params,
)(_argmax_body)


def _gather_social_body(ri_hbm, si_hbm, vid_hbm, pref_hbm, struct_hbm,
                        tp_hbm, ts_hbm, th_hbm,
                        riv, siv, piv, rowbuf_si, rowbuf_vid,
                        idxb, idxb2, vidb, pib, thw, thb, tpb, tsb,
                        pi_sh, sem_a, sem_b, sem_c):
    c = lax.axis_index("c")
    s = lax.axis_index("s")
    lane = lax.iota(jnp.int32, L)
    base = s * ROWS_PER_SUB
    m8 = lane < ROWS_PER_SUB

    @pl.when(c == 0)
    def _():
        pltpu.sync_copy(ri_hbm, riv.at[pl.ds(0, B)])
        pltpu.sync_copy(si_hbm, siv.at[pl.ds(0, B)])
        myri = riv[pl.ds(base, L)]
        mysi = siv[pl.ds(base, L)]
        idxb[...] = jnp.where(m8, myri + 100, 0)
        pltpu.async_copy(vid_hbm.at[idxb.at[pl.ds(0, ROWS_PER_SUB)]],
                         vidb.at[pl.ds(0, ROWS_PER_SUB)], sem_a).wait()
        vid16 = vidb[...]
        idxb[...] = jnp.where(m8, mysi, 0)
        idxb2[...] = jnp.where(m8, vid16, 0)
        cp1 = pltpu.async_copy(
            pref_hbm.at[idxb.at[pl.ds(0, ROWS_PER_SUB)]],
            rowbuf_si.at[pl.ds(0, ROWS_PER_SUB)], sem_a)
        cp2 = pltpu.async_copy(
            pref_hbm.at[idxb2.at[pl.ds(0, ROWS_PER_SUB)]],
            rowbuf_vid.at[pl.ds(0, ROWS_PER_SUB)], sem_b)
        th_copies = []
        for j in range(ROWS_PER_SUB):
            col0 = (myri[j] // L) * L
            cp = pltpu.async_copy(
                struct_hbm.at[mysi[j], pl.ds(col0, L)],
                thw.at[j], sem_c)
            th_copies.append(cp)
        cp1.wait()
        cp2.wait()
        for cp in th_copies:
            cp.wait()
        rowidx = jnp.where(m8, lane, 0)
        colidx = jnp.where(m8, base + lane, 0)
        pi16 = plsc.load_gather(rowbuf_si, [rowidx, colidx])
        pv16 = plsc.load_gather(rowbuf_vid, [rowidx, colidx])
        th16 = plsc.load_gather(
            thw, [rowidx, jnp.where(m8, myri % L, 0)])
        pib[...] = pi16
        tpb[...] = pi16 - pv16
        thb[...] = th16
        pltpu.sync_copy(tpb.at[pl.ds(0, ROWS_PER_SUB)],
                        tp_hbm.at[pl.ds(base, ROWS_PER_SUB)])
        pltpu.sync_copy(thb.at[pl.ds(0, ROWS_PER_SUB)],
                        th_hbm.at[pl.ds(base, ROWS_PER_SUB)])
        pltpu.sync_copy(pib.at[pl.ds(0, ROWS_PER_SUB)],
                        pi_sh.at[pl.ds(base, ROWS_PER_SUB)])

    plsc.subcore_barrier()

    @pl.when(c == 0)
    def _():
        pltpu.sync_copy(pi_sh, piv.at[pl.ds(0, B)])
        ts_vec = jnp.zeros((L,), jnp.float32)
        for j in range(ROWS_PER_SUB):
            i = base + j
            ivec = jnp.full((L,), i, jnp.int32)
            s_ib = plsc.load_gather(siv, [ivec])
            p_ib = plsc.load_gather(piv, [ivec])
            acc = jnp.zeros((L,), jnp.float32)
            for cc in range(B // L):
                sl = pl.ds(cc * L, L)
                sik = siv[sl]
                pik = piv[sl]
                idxv = lane + cc * L
                msk = jnp.logical_and(sik == s_ib, idxv != i)
                acc = acc + jnp.where(msk, jnp.minimum(pik, p_ib), 0.0)
            ts_j = jnp.sum(acc)
            ts_vec = jnp.where(lane == j, jnp.full((L,), ts_j, jnp.float32),
                               ts_vec)
        tsb[...] = ts_vec
        pltpu.sync_copy(tsb.at[pl.ds(0, ROWS_PER_SUB)],
                        ts_hbm.at[pl.ds(base, ROWS_PER_SUB)])


_gather_social_call = functools.partial(
    pl.kernel,
    out_type=(
        jax.ShapeDtypeStruct((B,), jnp.float32),
        jax.ShapeDtypeStruct((B,), jnp.float32),
        jax.ShapeDtypeStruct((B,), jnp.float32),
    ),
    mesh=_mesh,
    scratch_types=[
        pltpu.VMEM((B + L,), jnp.int32),
        pltpu.VMEM((B + L,), jnp.int32),
        pltpu.VMEM((B + L,), jnp.float32),
        pltpu.VMEM((ROWS_PER_SUB, B), jnp.float32),
        pltpu.VMEM((ROWS_PER_SUB, B), jnp.float32),
        pltpu.VMEM((L,), jnp.int32),
        pltpu.VMEM((L,), jnp.int32),
        pltpu.VMEM((L,), jnp.int32),
        pltpu.VMEM((L,), jnp.float32),
        pltpu.VMEM((ROWS_PER_SUB, L), jnp.float32),
        pltpu.VMEM((L,), jnp.float32),
        pltpu.VMEM((L,), jnp.float32),
        pltpu.VMEM((L,), jnp.float32),
        pltpu.VMEM_SHARED((B,), jnp.float32),
        pltpu.SemaphoreType.DMA,
        pltpu.SemaphoreType.DMA,
        pltpu.SemaphoreType.DMA,
    ],
    compiler_params=_params,
)(_gather_social_body)


def kernel(Recommended_m, Substitute_m, ItemGroups_m, Vid, VUU, KUU, Vscore,
           Kscore, preference, structure):
    del ItemGroups_m, VUU, KUU, Vscore, Kscore
    ri, si = _argmax_call(Recommended_m, Substitute_m)
    tp, ts, th = _gather_social_call(ri, si, Vid, preference, structure)
    return tp[:, None], ts[:, None], th[:, None]


# speedup vs baseline: 6.4749x; 1.0781x over previous
import functools

import jax
import jax.numpy as jnp
from jax import lax
from jax.experimental import pallas as pl
from jax.experimental.pallas import tpu as pltpu
from jax.experimental.pallas import tpu_sc as plsc

B = 128
N = 8192
M = 8192
L = 16
NSUB = 16
R = B // NSUB
CHUNKS = N // L
KACC = 4

_mesh = plsc.VectorSubcoreMesh(core_axis_name="c", subcore_axis_name="s",
                               num_cores=2, num_subcores=NSUB)
_params = pltpu.CompilerParams(needs_layout_passes=False)


def _row_argmax(row_ref):
    lane = lax.iota(jnp.int32, L)
    init = tuple(
        (jnp.full((L,), -jnp.inf, jnp.float32), jnp.zeros((L,), jnp.int32))
        for _ in range(KACC))

    def body(i, acc):
        out = []
        for k in range(KACC):
            cnum = i + k
            v = row_ref[pl.ds(cnum * L, L)]
            vm, vc = acc[k]
            gt = v > vm
            out.append((jnp.where(gt, v, vm),
                        jnp.where(gt, jnp.full((L,), cnum, jnp.int32), vc)))
        return tuple(out)

    acc = plsc.parallel_loop(0, CHUNKS, step=KACC, unroll=2, carry=init)(body)
    vm, vc = acc[0]
    for k in range(1, KACC):
        vm2, vc2 = acc[k]
        take = jnp.logical_or(vm2 > vm,
                              jnp.logical_and(vm2 == vm, vc2 < vc))
        vm = jnp.where(take, vm2, vm)
        vc = jnp.where(take, vc2, vc)
    m = jnp.max(vm)
    cand = jnp.where(vm == m, vc * L + lane, jnp.int32(2**30))
    return jnp.min(cand)


def _phase_a_body(rec_hbm, sub_hbm, vid_hbm, pref_hbm,
                  ri_hbm, si_hbm, vido_hbm, pio_hbm, ts_hbm,
                  buf0, buf1, resb, idxb, vidb, prefrows, pib,
                  siv, piv, tsb, si_sh, pi_sh, sem0, sem1, sem2):
    c = lax.axis_index("c")
    s = lax.axis_index("s")
    lane = lax.iota(jnp.int32, L)
    base = s * R
    m8 = lane < R
    bufs = [buf0, buf1]
    sems = [sem0, sem1]

    def argmax_rows(mat_hbm):
        res_vec = jnp.zeros((L,), jnp.int32)
        pltpu.async_copy(mat_hbm.at[base], bufs[0], sems[0])
        for j in range(R):
            if j + 1 < R:
                pltpu.async_copy(mat_hbm.at[base + j + 1], bufs[(j + 1) % 2],
                                 sems[(j + 1) % 2])
            pltpu.make_async_copy(mat_hbm.at[base + j], bufs[j % 2],
                                  sems[j % 2]).wait()
            am = _row_argmax(bufs[j % 2])
            res_vec = jnp.where(lane == j, jnp.full((L,), am, jnp.int32),
                                res_vec)
        return res_vec

    @pl.when(c == 0)
    def _():
        ri16 = argmax_rows(rec_hbm)
        resb[...] = ri16
        pltpu.sync_copy(resb.at[pl.ds(0, R)], ri_hbm.at[pl.ds(base, R)])
        idxb[...] = jnp.where(m8, ri16 + 100, 0)
        pltpu.async_copy(vid_hbm.at[idxb.at[pl.ds(0, R)]],
                         vidb.at[pl.ds(0, R)], sem2).wait()
        pltpu.sync_copy(vidb.at[pl.ds(0, R)], vido_hbm.at[pl.ds(base, R)])

    @pl.when(c == 1)
    def _():
        si16 = argmax_rows(sub_hbm)
        resb[...] = si16
        pltpu.sync_copy(resb.at[pl.ds(0, R)], si_hbm.at[pl.ds(base, R)])
        pltpu.sync_copy(resb.at[pl.ds(0, R)], si_sh.at[pl.ds(base, R)])
        idxb[...] = jnp.where(m8, si16, 0)
        pltpu.async_copy(pref_hbm.at[idxb.at[pl.ds(0, R)]],
                         prefrows.at[pl.ds(0, R)], sem2).wait()
        pi16 = plsc.load_gather(
            prefrows,
            [jnp.where(m8, lane, 0), jnp.where(m8, base + lane, 0)])
        pib[...] = pi16
        pltpu.sync_copy(pib.at[pl.ds(0, R)], pio_hbm.at[pl.ds(base, R)])
        pltpu.sync_copy(pib.at[pl.ds(0, R)], pi_sh.at[pl.ds(base, R)])

    plsc.subcore_barrier()

    @pl.when(c == 1)
    def _():
        pltpu.sync_copy(si_sh, siv.at[pl.ds(0, B)])
        pltpu.sync_copy(pi_sh, piv.at[pl.ds(0, B)])
        ts_vec = jnp.zeros((L,), jnp.float32)
        for j in range(R):
            i = base + j
            ivec = jnp.full((L,), i, jnp.int32)
            s_ib = plsc.load_gather(siv, [ivec])
            p_ib = plsc.load_gather(piv, [ivec])
            acc = jnp.zeros((L,), jnp.float32)
            for cc in range(B // L):
                sl = pl.ds(cc * L, L)
                sik = siv[sl]
                pik = piv[sl]
                idxv = lane + cc * L
                msk = jnp.logical_and(sik == s_ib, idxv != i)
                acc = acc + jnp.where(msk, jnp.minimum(pik, p_ib), 0.0)
            ts_j = jnp.sum(acc)
            ts_vec = jnp.where(lane == j, jnp.full((L,), ts_j, jnp.float32),
                               ts_vec)
        tsb[...] = ts_vec
        pltpu.sync_copy(tsb.at[pl.ds(0, R)], ts_hbm.at[pl.ds(base, R)])


_phase_a_call = functools.partial(
    pl.kernel,
    out_type=(
        jax.ShapeDtypeStruct((B,), jnp.int32),
        jax.ShapeDtypeStruct((B,), jnp.int32),
        jax.ShapeDtypeStruct((B,), jnp.int32),
        jax.ShapeDtypeStruct((B,), jnp.float32),
        jax.ShapeDtypeStruct((B,), jnp.float32),
    ),
    mesh=_mesh,
    scratch_types=[
        pltpu.VMEM((N,), jnp.float32),
        pltpu.VMEM((N,), jnp.float32),
        pltpu.VMEM((L,), jnp.int32),
        pltpu.VMEM((L,), jnp.int32),
        pltpu.VMEM((L,), jnp.int32),
        pltpu.VMEM((R, B), jnp.float32),
        pltpu.VMEM((L,), jnp.float32),
        pltpu.VMEM((B + L,), jnp.int32),
        pltpu.VMEM((B + L,), jnp.float32),
        pltpu.VMEM((L,), jnp.float32),
        pltpu.VMEM_SHARED((B,), jnp.int32),
        pltpu.VMEM_SHARED((B,), jnp.float32),
        pltpu.SemaphoreType.DMA,
        pltpu.SemaphoreType.DMA,
        pltpu.SemaphoreType.DMA,
    ],
    compiler_params=_params,
)(_phase_a_body)


def _phase_b_body(ri_hbm, si_hbm, vid_hbm, pi_hbm, pref_hbm, struct_hbm,
                  tp_hbm, th_hbm,
                  rb, sb, vb, pb, rowpv, thw, tpb, thb,
                  sem0, sem1, sem2, sem3):
    c = lax.axis_index("c")
    s = lax.axis_index("s")
    lane = lax.iota(jnp.int32, L)
    base = s * R
    m8 = lane < R

    @pl.when(c == 0)
    def _():
        cp_r = pltpu.async_copy(ri_hbm.at[pl.ds(base, R)],
                                rb.at[pl.ds(0, R)], sem0)
        cp_s = pltpu.async_copy(si_hbm.at[pl.ds(base, R)],
                                sb.at[pl.ds(0, R)], sem1)
        cp_v = pltpu.async_copy(vid_hbm.at[pl.ds(base, R)],
                                vb.at[pl.ds(0, R)], sem2)
        cp_p = pltpu.async_copy(pi_hbm.at[pl.ds(base, R)],
                                pb.at[pl.ds(0, R)], sem3)
        cp_r.wait()
        cp_s.wait()
        cp_v.wait()
        myri = rb[...]
        mysi = sb[...]
        cp_pv = pltpu.async_copy(pref_hbm.at[vb.at[pl.ds(0, R)]],
                                 rowpv.at[pl.ds(0, R)], sem0)
        th_copies = []
        for j in range(R):
            col0 = (myri[j] // L) * L
            cp = pltpu.async_copy(
                struct_hbm.at[mysi[j], pl.ds(col0, L)], thw.at[j], sem1)
            th_copies.append(cp)
        cp_pv.wait()
        for cp in th_copies:
            cp.wait()
        cp_p.wait()
        rowidx = jnp.where(m8, lane, 0)
        colidx = jnp.where(m8, base + lane, 0)
        pv16 = plsc.load_gather(rowpv, [rowidx, colidx])
        th16 = plsc.load_gather(thw, [rowidx, jnp.where(m8, myri % L, 0)])
        tpb[...] = pb[...] - pv16
        thb[...] = th16
        pltpu.sync_copy(tpb.at[pl.ds(0, R)], tp_hbm.at[pl.ds(base, R)])
        pltpu.sync_copy(thb.at[pl.ds(0, R)], th_hbm.at[pl.ds(base, R)])


_phase_b_call = functools.partial(
    pl.kernel,
    out_type=(
        jax.ShapeDtypeStruct((B,), jnp.float32),
        jax.ShapeDtypeStruct((B,), jnp.float32),
    ),
    mesh=_mesh,
    scratch_types=[
        pltpu.VMEM((L,), jnp.int32),
        pltpu.VMEM((L,), jnp.int32),
        pltpu.VMEM((L,), jnp.int32),
        pltpu.VMEM((L,), jnp.float32),
        pltpu.VMEM((R, B), jnp.float32),
        pltpu.VMEM((R, L), jnp.float32),
        pltpu.VMEM((L,), jnp.float32),
        pltpu.VMEM((L,), jnp.float32),
        pltpu.SemaphoreType.DMA,
        pltpu.SemaphoreType.DMA,
        pltpu.SemaphoreType.DMA,
        pltpu.SemaphoreType.DMA,
    ],
    compiler_params=_params,
)(_phase_b_body)


def kernel(Recommended_m, Substitute_m, ItemGroups_m, Vid, VUU, KUU, Vscore,
           Kscore, preference, structure):
    del ItemGroups_m, VUU, KUU, Vscore, Kscore
    ri, si, vid, pi, ts = _phase_a_call(Recommended_m, Substitute_m, Vid,
                                        preference)
    tp, th = _phase_b_call(ri, si, vid, pi, preference, structure)
    return tp[:, None], ts[:, None], th[:, None]


# speedup vs baseline: 8.6837x; 1.3411x over previous
import functools

import jax
import jax.numpy as jnp
from jax import lax
from jax.experimental import pallas as pl
from jax.experimental.pallas import tpu as pltpu
from jax.experimental.pallas import tpu_sc as plsc

B = 128
N = 8192
M = 8192
L = 16
NSUB = 16
R = B // NSUB

_mesh = plsc.VectorSubcoreMesh(core_axis_name="c", subcore_axis_name="s",
                               num_cores=2, num_subcores=NSUB)
_params = pltpu.CompilerParams(needs_layout_passes=False)


def _tc_argmax_body(rec_ref, sub_ref, ri_ref, si_ref):
    def row_argmax(x):
        m = jnp.max(x, axis=1, keepdims=True)
        iota = lax.broadcasted_iota(jnp.int32, x.shape, 1)
        return jnp.min(jnp.where(x == m, iota, jnp.int32(2**30)), axis=1)

    ri_ref[...] = row_argmax(rec_ref[...])
    si_ref[...] = row_argmax(sub_ref[...])


_tc_argmax = pl.pallas_call(
    _tc_argmax_body,
    out_shape=(
        jax.ShapeDtypeStruct((B,), jnp.int32),
        jax.ShapeDtypeStruct((B,), jnp.int32),
    ),
)


def _gather_social_body(ri_hbm, si_hbm, vid_hbm, pref_hbm, struct_hbm,
                        tp_hbm, ts_hbm, th_hbm,
                        rb, siv, piv, idxb, vidb, prefrows, rowpv,
                        thw, pib, tpb, thb, tsb, pi_sh,
                        sem0, sem1, sem2, sem3):
    c = lax.axis_index("c")
    s = lax.axis_index("s")
    lane = lax.iota(jnp.int32, L)
    base = s * R
    m8 = lane < R

    @pl.when(c == 0)
    def _():
        cp_r = pltpu.async_copy(ri_hbm.at[pl.ds(base, R)],
                                rb.at[pl.ds(0, R)], sem0)
        cp_s = pltpu.async_copy(si_hbm, siv.at[pl.ds(0, B)], sem1)
        cp_r.wait()
        cp_s.wait()
        myri = rb[...]
        mysi = siv[pl.ds(base, L)]
        idxb[...] = jnp.where(m8, myri + 100, 0)
        cp_v = pltpu.async_copy(vid_hbm.at[idxb.at[pl.ds(0, R)]],
                                vidb.at[pl.ds(0, R)], sem2)
        cp_pi = pltpu.async_copy(pref_hbm.at[siv.at[pl.ds(base, R)]],
                                 prefrows.at[pl.ds(0, R)], sem1)
        th_copies = []
        for j in range(R):
            col0 = (myri[j] // L) * L
            cp = pltpu.async_copy(
                struct_hbm.at[mysi[j], pl.ds(col0, L)], thw.at[j], sem3)
            th_copies.append(cp)
        cp_v.wait()
        cp_pv = pltpu.async_copy(pref_hbm.at[vidb.at[pl.ds(0, R)]],
                                 rowpv.at[pl.ds(0, R)], sem2)
        cp_pi.wait()
        rowidx = jnp.where(m8, lane, 0)
        colidx = jnp.where(m8, base + lane, 0)
        pi16 = plsc.load_gather(prefrows, [rowidx, colidx])
        pib[...] = pi16
        pltpu.sync_copy(pib.at[pl.ds(0, R)], pi_sh.at[pl.ds(base, R)])
        cp_pv.wait()
        for cp in th_copies:
            cp.wait()
        pv16 = plsc.load_gather(rowpv, [rowidx, colidx])
        th16 = plsc.load_gather(thw, [rowidx, jnp.where(m8, myri % L, 0)])
        tpb[...] = pi16 - pv16
        thb[...] = th16
        pltpu.sync_copy(tpb.at[pl.ds(0, R)], tp_hbm.at[pl.ds(base, R)])
        pltpu.sync_copy(thb.at[pl.ds(0, R)], th_hbm.at[pl.ds(base, R)])

    plsc.subcore_barrier()

    @pl.when(c == 0)
    def _():
        pltpu.sync_copy(pi_sh, piv.at[pl.ds(0, B)])
        ts_vec = jnp.zeros((L,), jnp.float32)
        for j in range(R):
            i = base + j
            ivec = jnp.full((L,), i, jnp.int32)
            s_ib = plsc.load_gather(siv, [ivec])
            p_ib = plsc.load_gather(piv, [ivec])
            acc = jnp.zeros((L,), jnp.float32)
            for cc in range(B // L):
                sl = pl.ds(cc * L, L)
                sik = siv[sl]
                pik = piv[sl]
                idxv = lane + cc * L
                msk = jnp.logical_and(sik == s_ib, idxv != i)
                acc = acc + jnp.where(msk, jnp.minimum(pik, p_ib), 0.0)
            ts_j = jnp.sum(acc)
            ts_vec = jnp.where(lane == j, jnp.full((L,), ts_j, jnp.float32),
                               ts_vec)
        tsb[...] = ts_vec
        pltpu.sync_copy(tsb.at[pl.ds(0, R)], ts_hbm.at[pl.ds(base, R)])


_gather_social_call = functools.partial(
    pl.kernel,
    out_type=(
        jax.ShapeDtypeStruct((B,), jnp.float32),
        jax.ShapeDtypeStruct((B,), jnp.float32),
        jax.ShapeDtypeStruct((B,), jnp.float32),
    ),
    mesh=_mesh,
    scratch_types=[
        pltpu.VMEM((L,), jnp.int32),
        pltpu.VMEM((B + L,), jnp.int32),
        pltpu.VMEM((B + L,), jnp.float32),
        pltpu.VMEM((L,), jnp.int32),
        pltpu.VMEM((L,), jnp.int32),
        pltpu.VMEM((R, B), jnp.float32),
        pltpu.VMEM((R, B), jnp.float32),
        pltpu.VMEM((R, L), jnp.float32),
        pltpu.VMEM((L,), jnp.float32),
        pltpu.VMEM((L,), jnp.float32),
        pltpu.VMEM((L,), jnp.float32),
        pltpu.VMEM((L,), jnp.float32),
        pltpu.VMEM_SHARED((B,), jnp.float32),
        pltpu.SemaphoreType.DMA,
        pltpu.SemaphoreType.DMA,
        pltpu.SemaphoreType.DMA,
        pltpu.SemaphoreType.DMA,
    ],
    compiler_params=_params,
)(_gather_social_body)


def kernel(Recommended_m, Substitute_m, ItemGroups_m, Vid, VUU, KUU, Vscore,
           Kscore, preference, structure):
    del ItemGroups_m, VUU, KUU, Vscore, Kscore
    ri, si = _tc_argmax(Recommended_m, Substitute_m)
    tp, ts, th = _gather_social_call(ri, si, Vid, preference, structure)
    return tp[:, None], ts[:, None], th[:, None]
